# baseline (speedup 1.0000x reference)
import functools

import jax
import jax.numpy as jnp
from jax import lax
from jax.experimental import pallas as pl
from jax.experimental.pallas import tpu as pltpu
from jax.experimental.pallas import tpu_sc as plsc

NC = 2
NS = 16
NW = NC * NS
L = 16


def _vgather16(vec, idx):
  return lax.gather(
      vec,
      idx[:, None],
      lax.GatherDimensionNumbers(
          offset_dims=(), collapsed_slice_dims=(0,), start_index_map=(0,)),
      slice_sizes=(1,),
      mode=lax.GatherScatterMode.PROMISE_IN_BOUNDS,
  )


def _make_sc_partials(n, ch, nchunk, rem):
  base = nchunk * ch

  def body(ftf_hbm, lab_hbm, cen_hbm, out_hbm,
           fxb0, fxb1, fyb0, fyb1, lb0, lb1, cbuf, asq, act, sem0, sem1):
    c = lax.axis_index("c")
    s = lax.axis_index("s")
    wid = s * NC + c
    sems = (sem0, sem1)
    fxbuf = (fxb0, fxb1)
    fybuf = (fyb0, fyb1)
    lbuf = (lb0, lb1)

    pltpu.sync_copy(cen_hbm, cbuf)
    zeros = jnp.zeros((L,), jnp.float32)
    ones = jnp.ones((L,), jnp.float32)
    iota = lax.iota(jnp.int32, L)
    lane17 = iota * 17
    for k in range(17):
      asq[pl.ds(16 * k, 16)] = zeros
      act[pl.ds(16 * k, 16)] = zeros
    cxv = cbuf[pl.ds(0, 16)]
    cyv = cbuf[pl.ds(16, 16)]

    def group(fxr, fyr, lr, g):
      lv = lr[pl.ds(g * 16, 16)]
      li = lv.astype(jnp.int32)
      fx = fxr[pl.ds(g * 16, 16)]
      fy = fyr[pl.ds(g * 16, 16)]
      cx = _vgather16(cxv, li)
      cy = _vgather16(cyv, li)
      dx = fx - cx
      dy = fy - cy
      sq = dx * dx + dy * dy
      sidx = lane17 + li
      plsc.addupdate_scatter(asq, [sidx], sq)
      plsc.addupdate_scatter(act, [sidx], ones)

    def start(k):
      cbase = wid * base + k * ch
      sl = k % 2
      return (
          pltpu.async_copy(lab_hbm.at[pl.ds(cbase, ch)], lbuf[sl], sems[sl]),
          pltpu.async_copy(ftf_hbm.at[pl.ds(cbase, ch)], fxbuf[sl],
                           sems[sl]),
          pltpu.async_copy(ftf_hbm.at[pl.ds(n + cbase, ch)], fybuf[sl],
                           sems[sl]),
      )

    handles = start(0)
    for k in range(nchunk):
      sl = k % 2
      nxt = start(k + 1) if k + 1 < nchunk else None
      for h in handles:
        h.wait()
      handles = nxt

      @plsc.parallel_loop(0, ch // 16, 1, unroll=16)
      def _grp(g):
        group(fxbuf[sl], fybuf[sl], lbuf[sl], g)

    if rem:
      @pl.when(wid == 0)
      def _tail():
        tbase = NW * base
        pltpu.sync_copy(lab_hbm.at[pl.ds(tbase, rem)],
                        lbuf[0].at[pl.ds(0, rem)])
        pltpu.sync_copy(ftf_hbm.at[pl.ds(tbase, rem)],
                        fxbuf[0].at[pl.ds(0, rem)])
        pltpu.sync_copy(ftf_hbm.at[pl.ds(n + tbase, rem)],
                        fybuf[0].at[pl.ds(0, rem)])

        def g_body(g, carry2):
          group(fxbuf[0], fybuf[0], lbuf[0], g)
          return carry2

        lax.fori_loop(0, rem // 16, g_body, 0)

    vsq = zeros
    vct = zeros
    for l in range(16):
      vsq = vsq + plsc.load_gather(asq, [iota + 17 * l])
      vct = vct + plsc.load_gather(act, [iota + 17 * l])
    cbuf[pl.ds(0, 16)] = vsq
    cbuf[pl.ds(16, 16)] = vct
    pltpu.sync_copy(cbuf.at[pl.ds(0, 16)], out_hbm.at[pl.ds(16 * wid, 16)])
    pltpu.sync_copy(cbuf.at[pl.ds(16, 16)],
                    out_hbm.at[pl.ds(16 * NW + 16 * wid, 16)])

  mesh = plsc.VectorSubcoreMesh(
      core_axis_name="c", subcore_axis_name="s", num_cores=NC,
      num_subcores=NS)
  return pl.kernel(
      body,
      out_type=jax.ShapeDtypeStruct((2 * 16 * NW,), jnp.float32),
      mesh=mesh,
      compiler_params=pltpu.CompilerParams(needs_layout_passes=False),
      scratch_types=[
          pltpu.VMEM((ch,), jnp.float32),
          pltpu.VMEM((ch,), jnp.float32),
          pltpu.VMEM((ch,), jnp.float32),
          pltpu.VMEM((ch,), jnp.float32),
          pltpu.VMEM((ch,), jnp.float32),
          pltpu.VMEM((ch,), jnp.float32),
          pltpu.VMEM((32,), jnp.float32),
          pltpu.VMEM((16 * 17,), jnp.float32),
          pltpu.VMEM((16 * 17,), jnp.float32),
          pltpu.SemaphoreType.DMA,
          pltpu.SemaphoreType.DMA,
      ],
  )


def _combine_kernel(n, part_ref, lam_ref, o_ref):
  x = part_ref[...]
  row = lax.broadcasted_iota(jnp.int32, (8, 128), 0)
  col = lax.broadcasted_iota(jnp.int32, (8, 128), 1)
  cls = col % 16
  is_sq = row < 4
  sq_x = jnp.where(is_sq, x, 0.0)
  ct_x = jnp.where(is_sq, 0.0, x)
  total = jnp.float32(0.0)
  for c in range(16):
    m = cls == c
    s_c = jnp.sum(jnp.where(m, sq_x, 0.0))
    t_c = jnp.sum(jnp.where(m, ct_x, 0.0))
    total += jnp.where(t_c > 0, s_c / jnp.maximum(t_c, 1.0), 0.0)
  loss = lam_ref[0, 0] * (0.5 / n) * total
  o_ref[...] = jnp.broadcast_to(loss, (1, 1))


def kernel(feature, label, lambdas, center):
  n = label.shape[0]
  nclass = center.shape[0]
  assert nclass <= 16
  base = (n // (NW * 8)) * 8
  rem = n - NW * base
  assert rem % 16 == 0
  ch = 16
  for d in range(12288, 15, -16):
    if base % d == 0:
      ch = d
      break
  nchunk = base // ch

  ftf = feature.T.reshape(-1)
  cen_cols = jnp.zeros((2, 16), jnp.float32).at[:, :nclass].set(
      center.T.astype(jnp.float32)).reshape(-1)

  partials = _make_sc_partials(n, ch, nchunk, rem)(ftf, label, cen_cols)

  lam = jnp.asarray(lambdas, jnp.float32).reshape(1, 1)
  loss = pl.pallas_call(
      functools.partial(_combine_kernel, n),
      out_shape=jax.ShapeDtypeStruct((1, 1), jnp.float32),
      in_specs=[
          pl.BlockSpec(memory_space=pltpu.VMEM),
          pl.BlockSpec(memory_space=pltpu.SMEM),
      ],
      out_specs=pl.BlockSpec(memory_space=pltpu.VMEM),
  )(partials.reshape(8, 128), lam)
  return loss[0, 0]


# speedup vs baseline: 1.0326x; 1.0326x over previous
import functools

import jax
import jax.numpy as jnp
from jax import lax
from jax.experimental import pallas as pl
from jax.experimental.pallas import tpu as pltpu
from jax.experimental.pallas import tpu_sc as plsc

NC = 2
NS = 16
NW = NC * NS
L = 16


def _vgather16(vec, idx):
  return lax.gather(
      vec,
      idx[:, None],
      lax.GatherDimensionNumbers(
          offset_dims=(), collapsed_slice_dims=(0,), start_index_map=(0,)),
      slice_sizes=(1,),
      mode=lax.GatherScatterMode.PROMISE_IN_BOUNDS,
  )


def _make_sc_partials(n, ch, nchunk, rem):
  base = nchunk * ch

  def body(ftf_hbm, lab_hbm, cen_hbm, out_hbm,
           fxb0, fxb1, fyb0, fyb1, lb0, lb1, cbuf, asq, act, sem0, sem1):
    c = lax.axis_index("c")
    s = lax.axis_index("s")
    wid = s * NC + c
    sems = (sem0, sem1)
    fxbuf = (fxb0, fxb1)
    fybuf = (fyb0, fyb1)
    lbuf = (lb0, lb1)

    pltpu.sync_copy(cen_hbm, cbuf)
    zeros = jnp.zeros((L,), jnp.float32)
    ones = jnp.ones((L,), jnp.float32)
    iota = lax.iota(jnp.int32, L)
    lane17 = iota * 17
    for k in range(17):
      asq[pl.ds(16 * k, 16)] = zeros
      act[pl.ds(16 * k, 16)] = zeros
    cxv = cbuf[pl.ds(0, 16)]
    cyv = cbuf[pl.ds(16, 16)]

    def group(fxr, fyr, lr, g):
      lv = lr[pl.ds(g * 16, 16)]
      li = lv.astype(jnp.int32)
      fx = fxr[pl.ds(g * 16, 16)]
      fy = fyr[pl.ds(g * 16, 16)]
      cx = _vgather16(cxv, li)
      cy = _vgather16(cyv, li)
      dx = fx - cx
      dy = fy - cy
      sq = dx * dx + dy * dy
      sidx = lane17 + li
      plsc.addupdate_scatter(asq, [sidx], sq)
      plsc.addupdate_scatter(act, [sidx], ones)

    def start(k):
      cbase = wid * base + k * ch
      sl = k % 2
      return (
          pltpu.async_copy(lab_hbm.at[pl.ds(cbase, ch)], lbuf[sl], sems[sl]),
          pltpu.async_copy(ftf_hbm.at[pl.ds(cbase, ch)], fxbuf[sl],
                           sems[sl]),
          pltpu.async_copy(ftf_hbm.at[pl.ds(n + cbase, ch)], fybuf[sl],
                           sems[sl]),
      )

    handles = start(0)
    for k in range(nchunk):
      sl = k % 2
      nxt = start(k + 1) if k + 1 < nchunk else None
      for h in handles:
        h.wait()
      handles = nxt

      @plsc.parallel_loop(0, ch // 16, 1, unroll=8)
      def _grp(g):
        group(fxbuf[sl], fybuf[sl], lbuf[sl], g)

    if rem:
      @pl.when(wid == 0)
      def _tail():
        tbase = NW * base
        pltpu.sync_copy(lab_hbm.at[pl.ds(tbase, rem)],
                        lbuf[0].at[pl.ds(0, rem)])
        pltpu.sync_copy(ftf_hbm.at[pl.ds(tbase, rem)],
                        fxbuf[0].at[pl.ds(0, rem)])
        pltpu.sync_copy(ftf_hbm.at[pl.ds(n + tbase, rem)],
                        fybuf[0].at[pl.ds(0, rem)])

        def g_body(g, carry2):
          group(fxbuf[0], fybuf[0], lbuf[0], g)
          return carry2

        lax.fori_loop(0, rem // 16, g_body, 0)

    vsq = zeros
    vct = zeros
    for l in range(16):
      vsq = vsq + plsc.load_gather(asq, [iota + 17 * l])
      vct = vct + plsc.load_gather(act, [iota + 17 * l])
    cbuf[pl.ds(0, 16)] = vsq
    cbuf[pl.ds(16, 16)] = vct
    pltpu.sync_copy(cbuf.at[pl.ds(0, 16)], out_hbm.at[pl.ds(16 * wid, 16)])
    pltpu.sync_copy(cbuf.at[pl.ds(16, 16)],
                    out_hbm.at[pl.ds(16 * NW + 16 * wid, 16)])

  mesh = plsc.VectorSubcoreMesh(
      core_axis_name="c", subcore_axis_name="s", num_cores=NC,
      num_subcores=NS)
  return pl.kernel(
      body,
      out_type=jax.ShapeDtypeStruct((2 * 16 * NW,), jnp.float32),
      mesh=mesh,
      compiler_params=pltpu.CompilerParams(
          needs_layout_passes=False, skip_device_barrier=True),
      scratch_types=[
          pltpu.VMEM((ch,), jnp.float32),
          pltpu.VMEM((ch,), jnp.float32),
          pltpu.VMEM((ch,), jnp.float32),
          pltpu.VMEM((ch,), jnp.float32),
          pltpu.VMEM((ch,), jnp.float32),
          pltpu.VMEM((ch,), jnp.float32),
          pltpu.VMEM((32,), jnp.float32),
          pltpu.VMEM((16 * 17,), jnp.float32),
          pltpu.VMEM((16 * 17,), jnp.float32),
          pltpu.SemaphoreType.DMA,
          pltpu.SemaphoreType.DMA,
      ],
  )


def _combine_kernel(n, part_ref, lam_ref, o_ref):
  x = part_ref[...]
  row = lax.broadcasted_iota(jnp.int32, (8, 128), 0)
  col = lax.broadcasted_iota(jnp.int32, (8, 128), 1)
  cls = col % 16
  is_sq = row < 4
  sq_x = jnp.where(is_sq, x, 0.0)
  ct_x = jnp.where(is_sq, 0.0, x)
  total = jnp.float32(0.0)
  for c in range(16):
    m = cls == c
    s_c = jnp.sum(jnp.where(m, sq_x, 0.0))
    t_c = jnp.sum(jnp.where(m, ct_x, 0.0))
    total += jnp.where(t_c > 0, s_c / jnp.maximum(t_c, 1.0), 0.0)
  loss = lam_ref[0, 0] * (0.5 / n) * total
  o_ref[...] = jnp.broadcast_to(loss, (1, 1))


def kernel(feature, label, lambdas, center):
  n = label.shape[0]
  nclass = center.shape[0]
  assert nclass <= 16
  base = (n // (NW * 8)) * 8
  rem = n - NW * base
  assert rem % 16 == 0
  ch = 16
  for d in range(12288, 15, -16):
    if base % d == 0:
      ch = d
      break
  nchunk = base // ch

  ftf = feature.T.reshape(-1)
  cen_cols = jnp.zeros((2, 16), jnp.float32).at[:, :nclass].set(
      center.T.astype(jnp.float32)).reshape(-1)

  partials = _make_sc_partials(n, ch, nchunk, rem)(ftf, label, cen_cols)

  lam = jnp.asarray(lambdas, jnp.float32).reshape(1, 1)
  loss = pl.pallas_call(
      functools.partial(_combine_kernel, n),
      out_shape=jax.ShapeDtypeStruct((1, 1), jnp.float32),
      in_specs=[
          pl.BlockSpec(memory_space=pltpu.VMEM),
          pl.BlockSpec(memory_space=pltpu.SMEM),
      ],
      out_specs=pl.BlockSpec(memory_space=pltpu.VMEM),
  )(partials.reshape(8, 128), lam)
  return loss[0, 0]
